# baseline (speedup 1.0000x reference)
import functools

import jax
import jax.numpy as jnp
from jax import lax
from jax.experimental import pallas as pl
from jax.experimental.pallas import tpu as pltpu
from jax.experimental.pallas import tpu_sc as plsc

BATCH = 16384
SEQ = 200
NBASIS = 8
RES = 256
SEQ_R = SEQ - 128
NHALF = 2
HB = BATCH // NHALF

TB = 2048


def _pre_body(amp_ref, basis_ref, pos_ref, idxl_ref, idxr_ref, tabl_ref, tabr_ref):
    p = pos_ref[...]
    idx = jnp.clip(((p + 1.0) * 127.5).astype(jnp.int32), 0, RES - 1)
    idxl_ref[...] = idx[:128, :].T
    idxr_ref[:, :SEQ_R] = idx[128:, :].T
    idxr_ref[:, SEQ_R:] = jnp.zeros((TB, 128 - SEQ_R), jnp.int32)
    amp = amp_ref[...]
    basis = basis_ref[...]
    cdims = (((0,), (0,)), ((), ()))
    tabl_ref[...] = lax.dot_general(
        amp, basis[:, :128], cdims, preferred_element_type=jnp.float32
    )
    tabr_ref[...] = lax.dot_general(
        amp, basis[:, 128:], cdims, preferred_element_type=jnp.float32
    )


def _pre_call(amp_t, hermite_basis, pos_t, half):
    n128 = jax.ShapeDtypeStruct((HB, 128), jnp.int32)
    f128 = jax.ShapeDtypeStruct((HB, 128), jnp.float32)
    off = half * (HB // TB)
    blk = lambda i: (i, 0)
    return pl.pallas_call(
        _pre_body,
        grid=(HB // TB,),
        in_specs=[
            pl.BlockSpec((NBASIS, TB), lambda i: (0, i + off)),
            pl.BlockSpec((NBASIS, RES), lambda i: (0, 0)),
            pl.BlockSpec((SEQ, TB), lambda i: (0, i + off)),
        ],
        out_specs=[
            pl.BlockSpec((TB, 128), blk),
            pl.BlockSpec((TB, 128), blk),
            pl.BlockSpec((TB, 128), blk),
            pl.BlockSpec((TB, 128), blk),
        ],
        out_shape=[n128, n128, f128, f128],
    )(amp_t, hermite_basis, pos_t)


def _post_body(outl0_ref, outr0_ref, outl1_ref, outr1_ref, out_ref):
    first = pl.program_id(0) < (HB // TB)
    outl = jnp.where(first, outl0_ref[...], outl1_ref[...])
    outr = jnp.where(first, outr0_ref[...], outr1_ref[...])
    out_ref[:128, :] = outl.T
    out_ref[128:, :] = outr.T[:SEQ_R, :]


def _post_call(outl0, outr0, outl1, outr1):
    nblk = HB // TB
    lo = lambda i: (jnp.minimum(i, nblk - 1), 0)
    hi = lambda i: (jnp.maximum(i - nblk, 0), 0)
    return pl.pallas_call(
        _post_body,
        grid=(BATCH // TB,),
        in_specs=[
            pl.BlockSpec((TB, 128), lo),
            pl.BlockSpec((TB, 128), lo),
            pl.BlockSpec((TB, 128), hi),
            pl.BlockSpec((TB, 128), hi),
        ],
        out_specs=pl.BlockSpec((SEQ, TB), lambda i: (0, i)),
        out_shape=jax.ShapeDtypeStruct((SEQ, BATCH), jnp.float32),
    )(outl0, outr0, outl1, outr1)


_INFO = plsc.get_sparse_core_info()
NC = _INFO.num_cores
NS = _INFO.num_subcores
NW = NC * NS
ROWS_PER_W = HB // NW
CHUNK = 64
N_CHUNKS = ROWS_PER_W // CHUNK
_LOG2_CHUNK = CHUNK.bit_length() - 1
_WINDOWS = 2 * CHUNK * 8


def _gather_body(
    idxl_hbm,
    idxr_hbm,
    tabl_hbm,
    tabr_hbm,
    outl_hbm,
    outr_hbm,
    idx_v,
    tab_v,
    res_v,
    ld0,
    ld1,
    st0,
    st1,
):
    wid = lax.axis_index("s") * NC + lax.axis_index("c")
    base = wid * ROWS_PER_W
    ld = (ld0, ld1)
    st = (st0, st1)

    def load_descs(ci, b):
        row0 = base + ci * CHUNK
        sl = pl.ds(row0, CHUNK)
        return (
            pltpu.make_async_copy(idxl_hbm.at[sl], idx_v.at[b, pl.ds(0, CHUNK)], ld[b]),
            pltpu.make_async_copy(idxr_hbm.at[sl], idx_v.at[b, pl.ds(CHUNK, CHUNK)], ld[b]),
            pltpu.make_async_copy(tabl_hbm.at[sl], tab_v.at[b, pl.ds(0, CHUNK)], ld[b]),
            pltpu.make_async_copy(tabr_hbm.at[sl], tab_v.at[b, pl.ds(CHUNK, CHUNK)], ld[b]),
        )

    def store_descs(ci, b):
        row0 = base + ci * CHUNK
        sl = pl.ds(row0, CHUNK)
        return (
            pltpu.make_async_copy(res_v.at[b, pl.ds(0, CHUNK)], outl_hbm.at[sl], st[b]),
            pltpu.make_async_copy(res_v.at[b, pl.ds(CHUNK, CHUNK)], outr_hbm.at[sl], st[b]),
        )

    for d in load_descs(0, 0):
        d.start()
    for ci in range(N_CHUNKS):
        b = ci % 2
        if ci + 1 < N_CHUNKS:
            for d in load_descs(ci + 1, 1 - b):
                d.start()
        for d in load_descs(ci, b):
            d.wait()
        if ci >= 2:
            for d in store_descs(ci - 2, b):
                d.wait()

        @plsc.parallel_loop(0, _WINDOWS, 1, unroll=8)
        def win_body(w):
            row = w >> 3
            col = (w & 7) * 16
            r = row & (CHUNK - 1)
            iv = idx_v[b, row, pl.ds(col, 16)]
            trow = ((iv >> 7) << _LOG2_CHUNK) + r
            tcol = iv & 127
            res_v[b, row, pl.ds(col, 16)] = plsc.load_gather(
                tab_v.at[b], [trow, tcol]
            )

        for d in store_descs(ci, b):
            d.start()
    for ci in (N_CHUNKS - 2, N_CHUNKS - 1):
        for d in store_descs(ci, ci % 2):
            d.wait()


_gather_call = functools.partial(
    pl.kernel,
    out_type=(
        jax.ShapeDtypeStruct((HB, 128), jnp.float32),
        jax.ShapeDtypeStruct((HB, 128), jnp.float32),
    ),
    mesh=plsc.VectorSubcoreMesh(core_axis_name="c", subcore_axis_name="s"),
    compiler_params=pltpu.CompilerParams(
        use_tc_tiling_on_sc=False, needs_layout_passes=False
    ),
    scratch_types=[
        pltpu.VMEM((2, 2 * CHUNK, 128), jnp.int32),
        pltpu.VMEM((2, 2 * CHUNK, 128), jnp.float32),
        pltpu.VMEM((2, 2 * CHUNK, 128), jnp.float32),
        pltpu.SemaphoreType.DMA,
        pltpu.SemaphoreType.DMA,
        pltpu.SemaphoreType.DMA,
        pltpu.SemaphoreType.DMA,
    ],
)(_gather_body)


def kernel(positions, amplitudes, hermite_basis):
    pos_t = positions.T
    amp_t = amplitudes.T
    idxl0, idxr0, tabl0, tabr0 = _pre_call(amp_t, hermite_basis, pos_t, 0)
    outl0, outr0 = _gather_call(idxl0, idxr0, tabl0, tabr0)
    idxl1, idxr1, tabl1, tabr1 = _pre_call(amp_t, hermite_basis, pos_t, 1)
    outl1, outr1 = _gather_call(idxl1, idxr1, tabl1, tabr1)
    return _post_call(outl0, outr0, outl1, outr1).T


# speedup vs baseline: 1.2145x; 1.2145x over previous
import functools

import jax
import jax.numpy as jnp
from jax import lax
from jax.experimental import pallas as pl
from jax.experimental.pallas import tpu as pltpu
from jax.experimental.pallas import tpu_sc as plsc

BATCH = 16384
SEQ = 200
NBASIS = 8
RES = 256
SEQ_R = SEQ - 128
NHALF = 2
HB = BATCH // NHALF

_MASK_LO = 0xFFFF
_MASK_HI = -65536

TB = 2048


def _bf16_bits(x):
    b16 = lax.bitcast_convert_type(x.astype(jnp.bfloat16), jnp.uint16)
    return b16.astype(jnp.int32)


def _pre_body(amp_ref, basis_ref, pos_ref, idx_ref, tab_ref):
    p = pos_ref[...]
    idx = jnp.clip(((p + 1.0) * 127.5).astype(jnp.int32), 0, RES - 1)
    idx_r = jnp.concatenate(
        [idx[128:, :], jnp.zeros((128 - SEQ_R, TB), jnp.int32)], axis=0
    )
    idx_ref[...] = (idx[:128, :] | (idx_r << 8)).T
    amp = amp_ref[...]
    basis = basis_ref[...]
    cdims = (((0,), (0,)), ((), ()))
    tabl = lax.dot_general(
        amp, basis[:, :128], cdims, preferred_element_type=jnp.float32
    )
    tabr = lax.dot_general(
        amp, basis[:, 128:], cdims, preferred_element_type=jnp.float32
    )
    tab_ref[...] = _bf16_bits(tabl) | (_bf16_bits(tabr) << 16)


def _pre_call(amp_t, hermite_basis, pos_t, half):
    n128 = jax.ShapeDtypeStruct((HB, 128), jnp.int32)
    off = half * (HB // TB)
    blk = lambda i: (i, 0)
    return pl.pallas_call(
        _pre_body,
        grid=(HB // TB,),
        in_specs=[
            pl.BlockSpec((NBASIS, TB), lambda i: (0, i + off)),
            pl.BlockSpec((NBASIS, RES), lambda i: (0, 0)),
            pl.BlockSpec((SEQ, TB), lambda i: (0, i + off)),
        ],
        out_specs=[pl.BlockSpec((TB, 128), blk), pl.BlockSpec((TB, 128), blk)],
        out_shape=[n128, n128],
    )(amp_t, hermite_basis, pos_t)


def _post_body(out0_ref, out1_ref, out_ref):
    first = pl.program_id(0) < (HB // TB)
    w = jnp.where(first, out0_ref[...], out1_ref[...])
    val_l = lax.bitcast_convert_type(w << 16, jnp.float32)
    val_r = lax.bitcast_convert_type(w & _MASK_HI, jnp.float32)
    out_ref[:128, :] = val_l.T
    out_ref[128:, :] = val_r.T[:SEQ_R, :]


def _post_call(out0, out1):
    nblk = HB // TB
    lo = lambda i: (jnp.minimum(i, nblk - 1), 0)
    hi = lambda i: (jnp.maximum(i - nblk, 0), 0)
    return pl.pallas_call(
        _post_body,
        grid=(BATCH // TB,),
        in_specs=[pl.BlockSpec((TB, 128), lo), pl.BlockSpec((TB, 128), hi)],
        out_specs=pl.BlockSpec((SEQ, TB), lambda i: (0, i)),
        out_shape=jax.ShapeDtypeStruct((SEQ, BATCH), jnp.float32),
    )(out0, out1)


_INFO = plsc.get_sparse_core_info()
NC = _INFO.num_cores
NS = _INFO.num_subcores
NW = NC * NS
ROWS_PER_W = HB // NW
CHUNK = 64
N_CHUNKS = ROWS_PER_W // CHUNK
_WINDOWS = CHUNK * 8


def _gather_body(
    idx_hbm,
    tab_hbm,
    out_hbm,
    idx_v,
    tab_v,
    res_v,
    ld0,
    ld1,
    st0,
    st1,
):
    wid = lax.axis_index("s") * NC + lax.axis_index("c")
    base = wid * ROWS_PER_W
    ld = (ld0, ld1)
    st = (st0, st1)

    def load_descs(ci, b):
        sl = pl.ds(base + ci * CHUNK, CHUNK)
        return (
            pltpu.make_async_copy(idx_hbm.at[sl], idx_v.at[b], ld[b]),
            pltpu.make_async_copy(tab_hbm.at[sl], tab_v.at[b], ld[b]),
        )

    def store_desc(ci, b):
        sl = pl.ds(base + ci * CHUNK, CHUNK)
        return pltpu.make_async_copy(res_v.at[b], out_hbm.at[sl], st[b])

    for d in load_descs(0, 0):
        d.start()
    for ci in range(N_CHUNKS):
        b = ci % 2
        if ci + 1 < N_CHUNKS:
            for d in load_descs(ci + 1, 1 - b):
                d.start()
        for d in load_descs(ci, b):
            d.wait()
        if ci >= 2:
            store_desc(ci - 2, b).wait()

        @plsc.parallel_loop(0, _WINDOWS, 1, unroll=8)
        def win_body(w):
            row = w >> 3
            col = (w & 7) * 16
            iv = idx_v[b, row, pl.ds(col, 16)]
            iv_l = iv & 255
            iv_r = (iv >> 8) & 255
            w_l = plsc.load_gather(tab_v.at[b], [jnp.full((16,), row, jnp.int32), iv_l & 127])
            w_r = plsc.load_gather(tab_v.at[b], [jnp.full((16,), row, jnp.int32), iv_r & 127])
            bits_l = jnp.where(iv_l > 127, (w_l >> 16) & _MASK_LO, w_l & _MASK_LO)
            bits_r = jnp.where(iv_r > 127, w_r & _MASK_HI, w_r << 16)
            res_v[b, row, pl.ds(col, 16)] = bits_l | bits_r

        store_desc(ci, b).start()
    for ci in (N_CHUNKS - 2, N_CHUNKS - 1):
        store_desc(ci, ci % 2).wait()


_gather_call = functools.partial(
    pl.kernel,
    out_type=jax.ShapeDtypeStruct((HB, 128), jnp.int32),
    mesh=plsc.VectorSubcoreMesh(core_axis_name="c", subcore_axis_name="s"),
    compiler_params=pltpu.CompilerParams(
        use_tc_tiling_on_sc=False, needs_layout_passes=False
    ),
    scratch_types=[
        pltpu.VMEM((2, CHUNK, 128), jnp.int32),
        pltpu.VMEM((2, CHUNK, 128), jnp.int32),
        pltpu.VMEM((2, CHUNK, 128), jnp.int32),
        pltpu.SemaphoreType.DMA,
        pltpu.SemaphoreType.DMA,
        pltpu.SemaphoreType.DMA,
        pltpu.SemaphoreType.DMA,
    ],
)(_gather_body)


def kernel(positions, amplitudes, hermite_basis):
    pos_t = positions.T
    amp_t = amplitudes.T
    idx0, tab0 = _pre_call(amp_t, hermite_basis, pos_t, 0)
    out0 = _gather_call(idx0, tab0)
    idx1, tab1 = _pre_call(amp_t, hermite_basis, pos_t, 1)
    out1 = _gather_call(idx1, tab1)
    return _post_call(out0, out1).T


# speedup vs baseline: 1.3044x; 1.0741x over previous
import functools

import jax
import jax.numpy as jnp
from jax import lax
from jax.experimental import pallas as pl
from jax.experimental.pallas import tpu as pltpu
from jax.experimental.pallas import tpu_sc as plsc

BATCH = 16384
SEQ = 200
NBASIS = 8
RES = 256
SEQ_R = SEQ - 128
NHALF = 2
HB = BATCH // NHALF

_MASK_LO = 0xFFFF
_MASK_HI = -65536

TB = 2048


def _bf16_bits(x):
    b16 = lax.bitcast_convert_type(x.astype(jnp.bfloat16), jnp.uint16)
    return b16.astype(jnp.int32)


def _pre_body(amp_ref, basis_ref, pos_ref, idx_ref, tab_ref):
    p = pos_ref[...]
    idx = jnp.clip(((p + 1.0) * 127.5).astype(jnp.int32), 0, RES - 1)
    idx_r = jnp.concatenate(
        [idx[128:, :], jnp.zeros((128 - SEQ_R, TB), jnp.int32)], axis=0
    )
    idx_ref[...] = (idx[:128, :] | (idx_r << 8)).T
    amp = amp_ref[...]
    basis = basis_ref[...]
    cdims = (((0,), (0,)), ((), ()))
    tabl = lax.dot_general(
        amp, basis[:, :128], cdims, preferred_element_type=jnp.float32
    )
    tabr = lax.dot_general(
        amp, basis[:, 128:], cdims, preferred_element_type=jnp.float32
    )
    tab_ref[...] = _bf16_bits(tabl) | (_bf16_bits(tabr) << 16)


def _pre_call(amp_t, hermite_basis, pos_t, half):
    n128 = jax.ShapeDtypeStruct((HB, 128), jnp.int32)
    off = half * (HB // TB)
    blk = lambda i: (i, 0)
    return pl.pallas_call(
        _pre_body,
        grid=(HB // TB,),
        in_specs=[
            pl.BlockSpec((NBASIS, TB), lambda i: (0, i + off)),
            pl.BlockSpec((NBASIS, RES), lambda i: (0, 0)),
            pl.BlockSpec((SEQ, TB), lambda i: (0, i + off)),
        ],
        out_specs=[pl.BlockSpec((TB, 128), blk), pl.BlockSpec((TB, 128), blk)],
        out_shape=[n128, n128],
    )(amp_t, hermite_basis, pos_t)


def _post0_body(outw_ref, out_ref):
    w = outw_ref[...]
    val_l = lax.bitcast_convert_type(w << 16, jnp.float32)
    val_r = lax.bitcast_convert_type(w & _MASK_HI, jnp.float32)
    out_ref[:128, :] = val_l.T
    out_ref[128:, :] = val_r.T[:SEQ_R, :]


def _post1_body(acc_ref, outw_ref, out_ref):
    del acc_ref
    _post0_body(outw_ref, out_ref)


def _post_call(out0, out1):
    nblk = HB // TB
    acc = pl.pallas_call(
        _post0_body,
        grid=(nblk,),
        in_specs=[pl.BlockSpec((TB, 128), lambda i: (i, 0))],
        out_specs=pl.BlockSpec((SEQ, TB), lambda i: (0, i)),
        out_shape=jax.ShapeDtypeStruct((SEQ, BATCH), jnp.float32),
    )(out0)
    return pl.pallas_call(
        _post1_body,
        grid=(nblk,),
        in_specs=[
            pl.BlockSpec(memory_space=pl.ANY),
            pl.BlockSpec((TB, 128), lambda i: (i, 0)),
        ],
        out_specs=pl.BlockSpec((SEQ, TB), lambda i: (0, i + nblk)),
        out_shape=jax.ShapeDtypeStruct((SEQ, BATCH), jnp.float32),
        input_output_aliases={0: 0},
    )(acc, out1)


_INFO = plsc.get_sparse_core_info()
NC = _INFO.num_cores
NS = _INFO.num_subcores
NW = NC * NS
ROWS_PER_W = HB // NW
CHUNK = 64
N_CHUNKS = ROWS_PER_W // CHUNK
_WINDOWS = CHUNK * 8


def _gather_body(
    idx_hbm,
    tab_hbm,
    out_hbm,
    idx_v,
    tab_v,
    res_v,
    ld0,
    ld1,
    st0,
    st1,
):
    wid = lax.axis_index("s") * NC + lax.axis_index("c")
    base = wid * ROWS_PER_W
    ld = (ld0, ld1)
    st = (st0, st1)

    def load_descs(ci, b):
        sl = pl.ds(base + ci * CHUNK, CHUNK)
        return (
            pltpu.make_async_copy(idx_hbm.at[sl], idx_v.at[b], ld[b]),
            pltpu.make_async_copy(tab_hbm.at[sl], tab_v.at[b], ld[b]),
        )

    def store_desc(ci, b):
        sl = pl.ds(base + ci * CHUNK, CHUNK)
        return pltpu.make_async_copy(res_v.at[b], out_hbm.at[sl], st[b])

    for d in load_descs(0, 0):
        d.start()
    for ci in range(N_CHUNKS):
        b = ci % 2
        if ci + 1 < N_CHUNKS:
            for d in load_descs(ci + 1, 1 - b):
                d.start()
        for d in load_descs(ci, b):
            d.wait()
        if ci >= 2:
            store_desc(ci - 2, b).wait()

        @plsc.parallel_loop(0, _WINDOWS, 1, unroll=8)
        def win_body(w):
            row = w >> 3
            col = (w & 7) * 16
            iv = idx_v[b, row, pl.ds(col, 16)]
            iv_l = iv & 255
            iv_r = (iv >> 8) & 255
            w_l = plsc.load_gather(tab_v.at[b], [jnp.full((16,), row, jnp.int32), iv_l & 127])
            w_r = plsc.load_gather(tab_v.at[b], [jnp.full((16,), row, jnp.int32), iv_r & 127])
            bits_l = jnp.where(iv_l > 127, (w_l >> 16) & _MASK_LO, w_l & _MASK_LO)
            bits_r = jnp.where(iv_r > 127, w_r & _MASK_HI, w_r << 16)
            res_v[b, row, pl.ds(col, 16)] = bits_l | bits_r

        store_desc(ci, b).start()
    for ci in (N_CHUNKS - 2, N_CHUNKS - 1):
        store_desc(ci, ci % 2).wait()


_gather_call = functools.partial(
    pl.kernel,
    out_type=jax.ShapeDtypeStruct((HB, 128), jnp.int32),
    mesh=plsc.VectorSubcoreMesh(core_axis_name="c", subcore_axis_name="s"),
    compiler_params=pltpu.CompilerParams(
        use_tc_tiling_on_sc=False, needs_layout_passes=False
    ),
    scratch_types=[
        pltpu.VMEM((2, CHUNK, 128), jnp.int32),
        pltpu.VMEM((2, CHUNK, 128), jnp.int32),
        pltpu.VMEM((2, CHUNK, 128), jnp.int32),
        pltpu.SemaphoreType.DMA,
        pltpu.SemaphoreType.DMA,
        pltpu.SemaphoreType.DMA,
        pltpu.SemaphoreType.DMA,
    ],
)(_gather_body)


def kernel(positions, amplitudes, hermite_basis):
    pos_t = positions.T
    amp_t = amplitudes.T
    idx0, tab0 = _pre_call(amp_t, hermite_basis, pos_t, 0)
    out0 = _gather_call(idx0, tab0)
    idx1, tab1 = _pre_call(amp_t, hermite_basis, pos_t, 1)
    out1 = _gather_call(idx1, tab1)
    return _post_call(out0, out1).T
